# baseline (speedup 1.0000x reference)
import functools

import jax
import jax.numpy as jnp
from jax import lax
from jax.experimental import pallas as pl
from jax.experimental.pallas import tpu as pltpu
from jax.experimental.pallas import tpu_sc as plsc

D = 64
CTX = 20
NC, NS, L = 2, 16, 16
NW = NC * NS
CB = 16
GSZ = 80
DEPTH = 4


def _sc_scores(pos_u, pos_v, neg_u, neg_v, table, lut):
    B = pos_v.shape[0]
    TB = 2 * B
    per_w = TB // NW
    n_chunks = per_w // CB
    nrow = CB * CTX
    nsplit = nrow // GSZ
    half = NW // 2

    mesh = plsc.VectorSubcoreMesh(
        core_axis_name="c", subcore_axis_name="s", num_cores=NC)

    @functools.partial(
        pl.kernel,
        out_type=jax.ShapeDtypeStruct((TB,), jnp.float32),
        mesh=mesh,
        scratch_types=[
            pltpu.VMEM((per_w * CTX,), jnp.int32),
            pltpu.VMEM((per_w,), jnp.int32),
            pltpu.VMEM((DEPTH, nrow, L), jnp.int32),
            pltpu.VMEM((DEPTH, CB, L), jnp.int32),
            pltpu.VMEM((L * CB,), jnp.float32),
            pltpu.VMEM((CB,), jnp.float32),
            pltpu.VMEM((256,), jnp.float32),
        ] + [pltpu.SemaphoreType.DMA] * DEPTH,
        compiler_params=pltpu.CompilerParams(
            needs_layout_passes=False, use_tc_tiling_on_sc=False
        ),
    )
    def k(pu_hbm, pv_hbm, nu_hbm, nv_hbm, tab_hbm, lut_hbm, out_hbm,
          rawc_v, rawv_v, crows_v, vrows_v, pbuf_v, sc_v, lut_v, *sems):
        wid = lax.axis_index("s") * NC + lax.axis_index("c")
        pltpu.sync_copy(lut_hbm, lut_v)

        @pl.when(wid < half)
        def _():
            pltpu.sync_copy(pu_hbm.at[pl.ds(wid * per_w * CTX, per_w * CTX)], rawc_v)
            pltpu.sync_copy(pv_hbm.at[pl.ds(wid * per_w, per_w)], rawv_v)

        @pl.when(wid >= half)
        def _():
            w2 = wid - half
            pltpu.sync_copy(nu_hbm.at[pl.ds(w2 * per_w * CTX, per_w * CTX)], rawc_v)
            pltpu.sync_copy(nv_hbm.at[pl.ds(w2 * per_w, per_w)], rawv_v)

        out_base = wid * per_w

        def copies(g, b):
            sem = sems[b]
            cps = [
                pltpu.make_async_copy(
                    tab_hbm.at[rawc_v.at[pl.ds(g * nrow + j * GSZ, GSZ)]],
                    crows_v.at[b, pl.ds(j * GSZ, GSZ)],
                    sem,
                )
                for j in range(nsplit)
            ]
            cps.append(pltpu.make_async_copy(
                tab_hbm.at[rawv_v.at[pl.ds(g * CB, CB)]], vrows_v.at[b], sem))
            return cps

        def fire(g, b):
            for cp in copies(g, b):
                cp.start()

        def drain(g, b):
            for cp in copies(g, b):
                cp.wait()

        for b in range(DEPTH):
            fire(b, b)

        lane = lax.iota(jnp.int32, L)

        def body(g, carry):
            bsel = lax.rem(g, DEPTH)

            for b in range(DEPTH):
                @pl.when(bsel == b)
                def _(b=b):
                    drain(g, b)

            def decode(xw):
                out = []
                for bp in range(4):
                    idx = lax.shift_right_logical(xw, 8 * bp) & 0xFF
                    out.append(plsc.load_gather(lut_v, [idx]))
                return out

            for i in range(CB):
                acc = decode(crows_v[bsel, i * CTX, pl.ds(0, L)])
                for c in range(1, CTX):
                    r = i * CTX + c
                    f = decode(crows_v[bsel, r, pl.ds(0, L)])
                    for kk in range(4):
                        acc[kk] = acc[kk] + f[kk]
                cf = decode(vrows_v[bsel, i, pl.ds(0, L)])
                p = acc[0] * cf[0]
                for kk in range(1, 4):
                    p = p + acc[kk] * cf[kk]
                plsc.store_scatter(pbuf_v, [lane * CB + i], p)

            for v in range(CB // L):
                sv = pbuf_v[pl.ds(v * L, L)]
                for d in range(1, L):
                    sv = sv + pbuf_v[pl.ds(d * CB + v * L, L)]
                sc_v[pl.ds(v * L, L)] = sv
            pltpu.sync_copy(sc_v, out_hbm.at[pl.ds(out_base + g * CB, CB)])

            for b in range(DEPTH):
                @pl.when(jnp.logical_and(g + DEPTH < n_chunks, bsel == b))
                def _(b=b):
                    fire(g + DEPTH, b)

            return carry

        lax.fori_loop(0, n_chunks, body, 0)

    return k(pos_u, pos_v, neg_u, neg_v, table, lut)


def _tc_loss(scores):
    n = scores.shape[0]
    x2 = scores.reshape(n // 128, 128)
    half_rows = n // 256

    def body(x_ref, o_ref):
        x = x_ref[...] * jnp.float32(2.0 ** -26)
        row = lax.broadcasted_iota(jnp.int32, x.shape, 0)
        y = jnp.where(row < half_rows, x, -x)
        ls = jnp.minimum(y, 0.0) - jnp.log1p(jnp.exp(-jnp.abs(y)))
        o_ref[0, 0] = -jnp.sum(ls)

    out = pl.pallas_call(
        body,
        out_shape=jax.ShapeDtypeStruct((1, 1), jnp.float32),
        out_specs=pl.BlockSpec(memory_space=pltpu.SMEM),
    )(x2)
    return out.reshape(())


def kernel(pos_u, pos_v, neg_u, neg_v, u_table, v_table):
    t8 = (u_table * jnp.float32(2.0 ** 13)).astype(jnp.float8_e4m3fn)
    bts = lax.bitcast_convert_type(t8, jnp.uint8).astype(jnp.uint32)
    q = D // 4
    w = (bts[:, 0:q] | (bts[:, q:2 * q] << 8)
         | (bts[:, 2 * q:3 * q] << 16) | (bts[:, 3 * q:4 * q] << 24))
    tab8 = lax.bitcast_convert_type(w, jnp.int32)
    lut = lax.bitcast_convert_type(
        jnp.arange(256, dtype=jnp.uint8), jnp.float8_e4m3fn
    ).astype(jnp.float32)
    scores = _sc_scores(
        pos_u.reshape(-1), pos_v, neg_u.reshape(-1), neg_v, tab8, lut)
    return _tc_loss(scores)


# speedup vs baseline: 2.3962x; 2.3962x over previous
import functools

import jax
import jax.numpy as jnp
from jax import lax
from jax.experimental import pallas as pl
from jax.experimental.pallas import tpu as pltpu
from jax.experimental.pallas import tpu_sc as plsc

D = 64
CTX = 20
NC, NS, L = 2, 16, 16
NW = NC * NS
CB = 16
GSZ = 80
DEPTH = 4


def _sc_scores(pos_u, pos_v, neg_u, neg_v, table):
    B = pos_v.shape[0]
    TB = 2 * B
    per_w = TB // NW
    n_chunks = per_w // CB
    nrow = CB * CTX
    nsplit = nrow // GSZ
    half = NW // 2

    mesh = plsc.VectorSubcoreMesh(
        core_axis_name="c", subcore_axis_name="s", num_cores=NC)

    @functools.partial(
        pl.kernel,
        out_type=jax.ShapeDtypeStruct((TB,), jnp.float32),
        mesh=mesh,
        scratch_types=[
            pltpu.VMEM((per_w * CTX,), jnp.int32),
            pltpu.VMEM((per_w,), jnp.int32),
            pltpu.VMEM((DEPTH, nrow, D), jnp.float32),
            pltpu.VMEM((DEPTH, CB, D), jnp.float32),
            pltpu.VMEM((L * CB,), jnp.float32),
            pltpu.VMEM((CB,), jnp.float32),
        ] + [pltpu.SemaphoreType.DMA] * DEPTH,
        compiler_params=pltpu.CompilerParams(
            needs_layout_passes=False, use_tc_tiling_on_sc=False
        ),
    )
    def k(pu_hbm, pv_hbm, nu_hbm, nv_hbm, tab_hbm, out_hbm,
          rawc_v, rawv_v, crows_v, vrows_v, pbuf_v, sc_v, *sems):
        wid = lax.axis_index("s") * NC + lax.axis_index("c")

        @pl.when(wid < half)
        def _():
            pltpu.sync_copy(pu_hbm.at[pl.ds(wid * per_w * CTX, per_w * CTX)], rawc_v)
            pltpu.sync_copy(pv_hbm.at[pl.ds(wid * per_w, per_w)], rawv_v)

        @pl.when(wid >= half)
        def _():
            w2 = wid - half
            pltpu.sync_copy(nu_hbm.at[pl.ds(w2 * per_w * CTX, per_w * CTX)], rawc_v)
            pltpu.sync_copy(nv_hbm.at[pl.ds(w2 * per_w, per_w)], rawv_v)

        out_base = wid * per_w

        def copies(g, b):
            sem = sems[b]
            cps = [
                pltpu.make_async_copy(
                    tab_hbm.at[rawc_v.at[pl.ds(g * nrow + j * GSZ, GSZ)]],
                    crows_v.at[b, pl.ds(j * GSZ, GSZ)],
                    sem,
                )
                for j in range(nsplit)
            ]
            cps.append(pltpu.make_async_copy(
                tab_hbm.at[rawv_v.at[pl.ds(g * CB, CB)]], vrows_v.at[b], sem))
            return cps

        def fire(g, b):
            for cp in copies(g, b):
                cp.start()

        def drain(g, b):
            for cp in copies(g, b):
                cp.wait()

        for b in range(DEPTH):
            fire(b, b)

        lane = lax.iota(jnp.int32, L)

        def body(g, carry):
            bsel = lax.rem(g, DEPTH)

            for b in range(DEPTH):
                @pl.when(bsel == b)
                def _(b=b):
                    drain(g, b)

            for i in range(CB):
                acc = [crows_v[bsel, i * CTX, pl.ds(kk * L, L)]
                       for kk in range(D // L)]
                for c in range(1, CTX):
                    r = i * CTX + c
                    for kk in range(D // L):
                        acc[kk] = acc[kk] + crows_v[bsel, r, pl.ds(kk * L, L)]
                p = acc[0] * vrows_v[bsel, i, pl.ds(0, L)]
                for kk in range(1, D // L):
                    p = p + acc[kk] * vrows_v[bsel, i, pl.ds(kk * L, L)]
                plsc.store_scatter(pbuf_v, [lane * CB + i], p)

            for v in range(CB // L):
                sv = pbuf_v[pl.ds(v * L, L)]
                for d in range(1, L):
                    sv = sv + pbuf_v[pl.ds(d * CB + v * L, L)]
                sc_v[pl.ds(v * L, L)] = sv
            pltpu.sync_copy(sc_v, out_hbm.at[pl.ds(out_base + g * CB, CB)])

            for b in range(DEPTH):
                @pl.when(jnp.logical_and(g + DEPTH < n_chunks, bsel == b))
                def _(b=b):
                    fire(g + DEPTH, b)

            return carry

        lax.fori_loop(0, n_chunks, body, 0)

    return k(pos_u, pos_v, neg_u, neg_v, table)


def _tc_loss(scores):
    n = scores.shape[0]
    x2 = scores.reshape(n // 128, 128)
    half_rows = n // 256

    def body(x_ref, o_ref):
        x = x_ref[...]
        row = lax.broadcasted_iota(jnp.int32, x.shape, 0)
        y = jnp.where(row < half_rows, x, -x)
        ls = jnp.minimum(y, 0.0) - jnp.log1p(jnp.exp(-jnp.abs(y)))
        o_ref[0, 0] = -jnp.sum(ls)

    out = pl.pallas_call(
        body,
        out_shape=jax.ShapeDtypeStruct((1, 1), jnp.float32),
        out_specs=pl.BlockSpec(memory_space=pltpu.SMEM),
    )(x2)
    return out.reshape(())


def kernel(pos_u, pos_v, neg_u, neg_v, u_table, v_table):
    scores = _sc_scores(
        pos_u.reshape(-1), pos_v, neg_u.reshape(-1), neg_v, u_table)
    return _tc_loss(scores)
